# baseline (speedup 1.0000x reference)
import functools

import jax
import jax.numpy as jnp
from jax.experimental import pallas as pl

_NUM_CLASSES = 81
_NEG_POS_RATIO = 3
_ALPHA = 1.0
_CHUNK = 2500


def _stage1_body(cls_ref, tgt_ref, regp_ref, regt_ref,
                 ce_ref, npos_ref, posce_ref, loc_ref, *, nc):
    x = cls_ref[0]
    tgt = tgt_ref[0]
    m = jnp.max(x, axis=-1, keepdims=True)
    e = jnp.exp(x - m)
    s = jnp.sum(e, axis=-1, keepdims=True)
    lse = m + jnp.log(s)
    lane = jax.lax.broadcasted_iota(jnp.int32, x.shape, 1)
    picked = jnp.sum(jnp.where(lane == tgt, x, 0.0), axis=-1, keepdims=True)
    ce = lse - picked
    pos = tgt > 0
    posf = pos.astype(jnp.float32)
    ce_ref[0] = jnp.where(pos, 0.0, ce)

    npos_p = jnp.sum(posf)
    posce_p = jnp.sum(ce * posf)
    d = regp_ref[0] - regt_ref[0]
    ad = jnp.abs(d)
    sl1 = jnp.where(ad < 1.0, 0.5 * ad * ad, ad - 0.5)
    loc_p = jnp.sum(sl1 * posf)

    i = pl.program_id(0)
    zero = jnp.zeros((1, 1, 1), jnp.float32)

    @pl.when(i % nc == 0)
    def _init():
        npos_ref[...] = zero
        posce_ref[...] = zero
        loc_ref[...] = zero

    npos_ref[...] += npos_p.reshape(1, 1, 1)
    posce_ref[...] += posce_p.reshape(1, 1, 1)
    loc_ref[...] += loc_p.reshape(1, 1, 1)


def _stage2_body(ce_ref, npos_ref, posce_ref, loc_ref, out_ref, *, n):
    v = ce_ref[...]
    bits = jax.lax.bitcast_convert_type(v, jnp.int32)
    npos = npos_ref[...][:, :, 0]
    k = jnp.minimum(_NEG_POS_RATIO * npos, float(n - 1))

    def step(t, lo):
        cand = lo | (1 << (30 - t))
        cnt = jnp.sum((bits >= cand).astype(jnp.float32), axis=1,
                      keepdims=True)
        return jnp.where(cnt >= k, cand, lo)

    lo = jax.lax.fori_loop(0, 31, step, jnp.zeros(k.shape, jnp.int32))
    t = jax.lax.bitcast_convert_type(lo, jnp.float32)
    gt = bits > lo
    c_gt = jnp.sum(gt.astype(jnp.float32), axis=1, keepdims=True)
    s_gt = jnp.sum(jnp.where(gt, v, 0.0), axis=1, keepdims=True)
    top = jnp.where(k > 0, s_gt + (k - c_gt) * t, 0.0)

    cls_loss = jnp.sum(posce_ref[...]) + jnp.sum(top)
    loc_loss = jnp.sum(loc_ref[...])
    npos_tot = jnp.sum(npos)
    denom = jnp.where(npos_tot > 0.0, npos_tot, 1.0)
    loc_n = _ALPHA * loc_loss / denom
    cls_n = cls_loss / denom
    total = jnp.where(npos_tot > 0.0, cls_n + loc_n, 0.0)
    lane4 = jax.lax.broadcasted_iota(jnp.int32, (1, 4), 1)
    out_ref[...] = jnp.where(
        lane4 == 0, total,
        jnp.where(lane4 == 1, cls_n, jnp.where(lane4 == 2, loc_n, 0.0)))


def _run(cls_preds, reg_preds, cls_targets, reg_targets, interpret=False):
    b, n, c = cls_preds.shape
    nc = n // _CHUNK
    g = b * nc

    cls_r = cls_preds.reshape(g, _CHUNK, c)
    tgt_r = cls_targets.reshape(g, _CHUNK, 1)
    regp_r = reg_preds.reshape(g, _CHUNK, 4)
    regt_r = reg_targets.reshape(g, _CHUNK, 4)

    ce_neg, npos, posce, loc = pl.pallas_call(
        functools.partial(_stage1_body, nc=nc),
        grid=(g,),
        in_specs=[
            pl.BlockSpec((1, _CHUNK, c), lambda i: (i, 0, 0)),
            pl.BlockSpec((1, _CHUNK, 1), lambda i: (i, 0, 0)),
            pl.BlockSpec((1, _CHUNK, 4), lambda i: (i, 0, 0)),
            pl.BlockSpec((1, _CHUNK, 4), lambda i: (i, 0, 0)),
        ],
        out_specs=[
            pl.BlockSpec((1, _CHUNK, 1), lambda i: (i, 0, 0)),
            pl.BlockSpec((1, 1, 1), lambda i: (i // nc, 0, 0)),
            pl.BlockSpec((1, 1, 1), lambda i: (i // nc, 0, 0)),
            pl.BlockSpec((1, 1, 1), lambda i: (i // nc, 0, 0)),
        ],
        out_shape=[
            jax.ShapeDtypeStruct((g, _CHUNK, 1), jnp.float32),
            jax.ShapeDtypeStruct((b, 1, 1), jnp.float32),
            jax.ShapeDtypeStruct((b, 1, 1), jnp.float32),
            jax.ShapeDtypeStruct((b, 1, 1), jnp.float32),
        ],
        interpret=interpret,
    )(cls_r, tgt_r, regp_r, regt_r)

    out = pl.pallas_call(
        functools.partial(_stage2_body, n=n),
        out_shape=jax.ShapeDtypeStruct((1, 4), jnp.float32),
        interpret=interpret,
    )(ce_neg.reshape(b, n), npos, posce, loc)

    return (out[0, 0], out[0, 1], out[0, 2])


@jax.jit
def kernel(cls_preds, reg_preds, cls_targets, reg_targets):
    return _run(cls_preds, reg_preds, cls_targets, reg_targets)
